# baseline (speedup 1.0000x reference)
import functools

import jax
import jax.numpy as jnp
from jax import lax
from jax.experimental import pallas as pl
from jax.experimental.pallas import tpu as pltpu
from jax.experimental.pallas import tpu_sc as plsc

NC = 2
NS = 16
NW = NC * NS
CHUNK = 128
NBUF = 4


def _deg_body(n, npad, ch, ei_hbm, deg_hbm, rowsg_hbm,
              rows_v, cols_v, val_v, zeros_v, deg_sp, sem_s):
  cid = lax.axis_index("c")
  sid = lax.axis_index("s")
  wid = sid * NC + cid
  rpt = npad // NS
  dump = npad - n

  def _z(i, c):
    zeros_v[pl.ds(i * 16, 16)] = jnp.zeros((16,), jnp.float32)
    return c
  lax.fori_loop(0, rpt // 16, _z, 0)
  pltpu.sync_copy(zeros_v, deg_sp.at[pl.ds(sid * rpt, rpt)])

  pltpu.sync_copy(ei_hbm.at[0, pl.ds(wid * ch, ch)], rows_v)
  pltpu.sync_copy(ei_hbm.at[1, pl.ds(wid * ch, ch)], cols_v)
  plsc.subcore_barrier()

  lanes = lax.iota(jnp.int32, 16)

  def _chunk(k, c):
    for j in range(CHUNK // 16):
      sl = pl.ds(j * 16, 16)
      r = rows_v[k, sl]
      cc = cols_v[k, sl]
      m = r != cc
      base = (wid * ch + k) * CHUNK + j * 16
      spr = n + ((base + lanes) % dump)
      rows_v[k, sl] = jnp.where(m, r, spr)
      val_v[k, sl] = jnp.where(m, 1.0, 0.0)
    pltpu.async_copy(val_v.at[k], deg_sp.at[cols_v.at[k]], sem_s, add=True)
    return c
  lax.fori_loop(0, ch, _chunk, 0)

  pltpu.sync_copy(rows_v, rowsg_hbm.at[pl.ds(wid * ch, ch)])

  def _drain(k, c):
    pltpu.make_async_copy(val_v.at[k], deg_sp.at[cols_v.at[k]], sem_s).wait()
    return c
  lax.fori_loop(0, ch, _drain, 0)

  plsc.subcore_barrier()
  pltpu.sync_copy(deg_sp.at[pl.ds(sid * rpt, rpt)],
                  deg_hbm.at[cid, 0, pl.ds(sid * rpt, rpt)])


def _spmm_body(npad, ch, fw, table_hbm, rows_hbm, ei_hbm, part_hbm,
               rows_v, cols_v, gbuf, acc_sp, sem_g, sem_s):
  cid = lax.axis_index("c")
  sid = lax.axis_index("s")
  wid = sid * NC + cid
  rpt = npad // NS
  r0 = sid * rpt

  def _z(i, c):
    for j in range(fw // 16):
      gbuf[0, 0, i, pl.ds(j * 16, 16)] = jnp.zeros((16,), jnp.float32)
    return c
  lax.fori_loop(0, CHUNK, _z, 0)
  for b in range(rpt // CHUNK):
    pltpu.sync_copy(gbuf.at[0, 0], acc_sp.at[pl.ds(r0 + b * CHUNK, CHUNK)])

  pltpu.sync_copy(rows_hbm.at[pl.ds(wid * ch, ch)], rows_v)
  pltpu.sync_copy(ei_hbm.at[1, pl.ds(wid * ch, ch)], cols_v)

  def _gathers(g, bank):
    for j in range(NBUF):
      pltpu.async_copy(table_hbm.at[rows_v.at[g * NBUF + j]],
                       gbuf.at[bank, j], sem_g)

  def _gathers_wait(g, bank):
    for j in range(NBUF):
      pltpu.make_async_copy(table_hbm.at[rows_v.at[g * NBUF + j]],
                            gbuf.at[bank, j], sem_g).wait()

  def _scatters(g, bank):
    for j in range(NBUF):
      pltpu.async_copy(gbuf.at[bank, j],
                       acc_sp.at[cols_v.at[g * NBUF + j]], sem_s, add=True)

  def _scatters_wait(g, bank):
    for j in range(NBUF):
      pltpu.make_async_copy(gbuf.at[bank, j],
                            acc_sp.at[cols_v.at[g * NBUF + j]], sem_s).wait()

  npairs = ch // NBUF // 2
  plsc.subcore_barrier()
  _gathers(0, 0)

  def _pair(p, c):
    a = 2 * p
    _gathers(a + 1, 1)
    _gathers_wait(a, 0)
    _scatters(a, 0)
    _scatters_wait(a, 0)

    @pl.when(p < npairs - 1)
    def _():
      _gathers(a + 2, 0)

    _gathers_wait(a + 1, 1)
    _scatters(a + 1, 1)
    _scatters_wait(a + 1, 1)
    return c
  lax.fori_loop(0, npairs, _pair, 0)

  plsc.subcore_barrier()
  pltpu.sync_copy(acc_sp.at[pl.ds(r0, rpt)],
                  part_hbm.at[cid, pl.ds(r0, rpt)])


def _mm1_body(x_ref, w_ref, y_ref):
  y_ref[...] = jnp.dot(x_ref[...], w_ref[...],
                       preferred_element_type=jnp.float32)


def _scale_body(y_ref, disb_ref, t_ref):
  t_ref[...] = disb_ref[...] * y_ref[...]


def _tc2_body(part_ref, t1_ref, disb_ref, b1_ref, w2_ref, t2_ref):
  agg = disb_ref[...] * (part_ref[0] + part_ref[1] + t1_ref[...])
  h = jnp.maximum(agg + b1_ref[...], 0.0)
  y2 = jnp.dot(h, w2_ref[...], preferred_element_type=jnp.float32)
  t2_ref[...] = disb_ref[...] * y2


def _tc3_body(fw, part_ref, t2_ref, disb_ref, b2_ref, out_ref):
  o = disb_ref[...] * (part_ref[0] + part_ref[1] + t2_ref[...]) + b2_ref[...]
  for h in range(2):
    oh = o[:, h * fw:(h + 1) * fw]
    m = jnp.max(oh, axis=-1, keepdims=True)
    lse = jnp.log(jnp.sum(jnp.exp(oh - m), axis=-1, keepdims=True)) + m
    out_ref[:, pl.ds(h * fw, fw)] = oh - lse


def kernel(x, edge_index, W1, b1, W2, b2):
  n, f_in = x.shape
  h_dim = W1.shape[1]
  c_dim = W2.shape[1]
  e = edge_index.shape[1]

  npad = ((n + 511) // 512 + (1 if n % 512 == 0 else 0)) * 512
  ch = -(-e // (NW * CHUNK))
  ch = -(-ch // 8) * 8
  e_pad = ch * NW * CHUNK
  blk = 1024
  blk2 = blk // 2
  grid = npad // blk

  ei_pad = jnp.pad(edge_index, ((0, 0), (0, e_pad - e)))
  ei_pad = ei_pad.reshape(2, e_pad // CHUNK, CHUNK)

  mesh = plsc.VectorSubcoreMesh(core_axis_name="c", subcore_axis_name="s")
  scp = pltpu.CompilerParams(use_tc_tiling_on_sc=False)

  deg_call = pl.kernel(
      functools.partial(_deg_body, n, npad, ch),
      out_type=(
          jax.ShapeDtypeStruct((NC, 1, npad), jnp.float32),
          jax.ShapeDtypeStruct((e_pad // CHUNK, CHUNK), jnp.int32),
      ),
      mesh=mesh,
      scratch_types=[
          pltpu.VMEM((ch, CHUNK), jnp.int32),
          pltpu.VMEM((ch, CHUNK), jnp.int32),
          pltpu.VMEM((ch, CHUNK), jnp.float32),
          pltpu.VMEM((npad // NS,), jnp.float32),
          pltpu.VMEM_SHARED((npad,), jnp.float32),
          pltpu.SemaphoreType.DMA,
      ],
      compiler_params=scp,
  )

  def spmm_call(fw):
    return pl.kernel(
        functools.partial(_spmm_body, npad, ch, fw),
        out_type=jax.ShapeDtypeStruct((NC, npad, fw), jnp.float32),
        mesh=mesh,
        scratch_types=[
            pltpu.VMEM((ch, CHUNK), jnp.int32),
            pltpu.VMEM((ch, CHUNK), jnp.int32),
            pltpu.VMEM((2, NBUF, CHUNK, fw), jnp.float32),
            pltpu.VMEM_SHARED((npad, fw), jnp.float32),
            pltpu.SemaphoreType.DMA,
            pltpu.SemaphoreType.DMA,
        ],
        compiler_params=scp,
    )

  spmm1 = spmm_call(h_dim)
  spmm2 = spmm1 if c_dim == h_dim else spmm_call(c_dim)

  deg_part, rows2d = deg_call(ei_pad)

  deg = deg_part[0, 0] + deg_part[1, 0] + 1.0
  dis = jnp.where(jnp.arange(npad) < n, lax.rsqrt(deg), 0.0)
  disb = jnp.repeat(dis.reshape(npad // 2, 2), h_dim, axis=1)

  y1 = pl.pallas_call(
      _mm1_body,
      grid=(grid,),
      in_specs=[
          pl.BlockSpec((blk, f_in), lambda i: (i, 0)),
          pl.BlockSpec((f_in, h_dim), lambda i: (0, 0)),
      ],
      out_specs=pl.BlockSpec((blk, h_dim), lambda i: (i, 0)),
      out_shape=jax.ShapeDtypeStruct((npad, h_dim), jnp.float32),
  )(x, W1)

  pk = (blk2, 2 * h_dim)
  t1p = pl.pallas_call(
      _scale_body,
      grid=(grid,),
      in_specs=[
          pl.BlockSpec(pk, lambda i: (i, 0)),
          pl.BlockSpec(pk, lambda i: (i, 0)),
      ],
      out_specs=pl.BlockSpec(pk, lambda i: (i, 0)),
      out_shape=jax.ShapeDtypeStruct((npad // 2, 2 * h_dim), jnp.float32),
  )(y1.reshape(npad // 2, 2 * h_dim), disb)

  t1 = t1p.reshape(npad, h_dim)
  part1 = spmm1(t1, rows2d, ei_pad)

  w2d = jnp.zeros((2 * h_dim, 2 * c_dim), jnp.float32)
  w2d = w2d.at[:h_dim, :c_dim].set(W2).at[h_dim:, c_dim:].set(W2)
  b1p = jnp.concatenate([b1, b1]).reshape(1, 2 * h_dim)
  b2p = jnp.concatenate([b2, b2]).reshape(1, 2 * c_dim)

  t2p = pl.pallas_call(
      _tc2_body,
      grid=(grid,),
      in_specs=[
          pl.BlockSpec((NC, blk2, 2 * h_dim), lambda i: (0, i, 0)),
          pl.BlockSpec(pk, lambda i: (i, 0)),
          pl.BlockSpec(pk, lambda i: (i, 0)),
          pl.BlockSpec((1, 2 * h_dim), lambda i: (0, 0)),
          pl.BlockSpec((2 * h_dim, 2 * c_dim), lambda i: (0, 0)),
      ],
      out_specs=pl.BlockSpec((blk2, 2 * c_dim), lambda i: (i, 0)),
      out_shape=jax.ShapeDtypeStruct((npad // 2, 2 * c_dim), jnp.float32),
  )(part1.reshape(NC, npad // 2, 2 * h_dim), t1p, disb, b1p, w2d)

  part2 = spmm2(t2p.reshape(npad, c_dim), rows2d, ei_pad)

  outp = pl.pallas_call(
      functools.partial(_tc3_body, c_dim),
      grid=(grid,),
      in_specs=[
          pl.BlockSpec((NC, blk2, 2 * c_dim), lambda i: (0, i, 0)),
          pl.BlockSpec((blk2, 2 * c_dim), lambda i: (i, 0)),
          pl.BlockSpec((blk2, 2 * c_dim), lambda i: (i, 0)),
          pl.BlockSpec((1, 2 * c_dim), lambda i: (0, 0)),
      ],
      out_specs=pl.BlockSpec((blk2, 2 * c_dim), lambda i: (i, 0)),
      out_shape=jax.ShapeDtypeStruct((npad // 2, 2 * c_dim), jnp.float32),
  )(part2.reshape(NC, npad // 2, 2 * c_dim), t2p, disb, b2p)

  return outp.reshape(npad, c_dim)[:n]


# speedup vs baseline: 1.0080x; 1.0080x over previous
import functools

import jax
import jax.numpy as jnp
from jax import lax
from jax.experimental import pallas as pl
from jax.experimental.pallas import tpu as pltpu
from jax.experimental.pallas import tpu_sc as plsc

NC = 2
NS = 16
NW = NC * NS
CHUNK = 128
NBUF = 2


def _deg_body(n, npad, ch, ei_hbm, deg_hbm, rowsg_hbm,
              rows_v, cols_v, val_v, zeros_v, deg_sp, sem_s):
  cid = lax.axis_index("c")
  sid = lax.axis_index("s")
  wid = sid * NC + cid
  rpt = npad // NS
  dump = npad - n

  def _z(i, c):
    zeros_v[pl.ds(i * 16, 16)] = jnp.zeros((16,), jnp.float32)
    return c
  lax.fori_loop(0, rpt // 16, _z, 0)
  pltpu.sync_copy(zeros_v, deg_sp.at[pl.ds(sid * rpt, rpt)])

  pltpu.sync_copy(ei_hbm.at[0, pl.ds(wid * ch, ch)], rows_v)
  pltpu.sync_copy(ei_hbm.at[1, pl.ds(wid * ch, ch)], cols_v)
  plsc.subcore_barrier()

  lanes = lax.iota(jnp.int32, 16)

  def _chunk(k, c):
    for j in range(CHUNK // 16):
      sl = pl.ds(j * 16, 16)
      r = rows_v[k, sl]
      cc = cols_v[k, sl]
      m = r != cc
      base = (wid * ch + k) * CHUNK + j * 16
      spr = n + ((base + lanes) % dump)
      rows_v[k, sl] = jnp.where(m, r, spr)
      val_v[k, sl] = jnp.where(m, 1.0, 0.0)
    pltpu.async_copy(val_v.at[k], deg_sp.at[cols_v.at[k]], sem_s, add=True)
    return c
  lax.fori_loop(0, ch, _chunk, 0)

  pltpu.sync_copy(rows_v, rowsg_hbm.at[pl.ds(wid * ch, ch)])

  def _drain(k, c):
    pltpu.make_async_copy(val_v.at[k], deg_sp.at[cols_v.at[k]], sem_s).wait()
    return c
  lax.fori_loop(0, ch, _drain, 0)

  plsc.subcore_barrier()
  pltpu.sync_copy(deg_sp.at[pl.ds(sid * rpt, rpt)],
                  deg_hbm.at[cid, 0, pl.ds(sid * rpt, rpt)])


def _spmm_body(npad, ch, fw, table_hbm, rows_hbm, ei_hbm, part_hbm,
               rows_v, cols_v, gbuf, acc_sp, sem_g, sem_s):
  cid = lax.axis_index("c")
  sid = lax.axis_index("s")
  wid = sid * NC + cid
  rpt = npad // NS
  r0 = sid * rpt

  def _z(i, c):
    for j in range(fw // 16):
      gbuf[0, 0, i, pl.ds(j * 16, 16)] = jnp.zeros((16,), jnp.float32)
    return c
  lax.fori_loop(0, CHUNK, _z, 0)
  for b in range(rpt // CHUNK):
    pltpu.sync_copy(gbuf.at[0, 0], acc_sp.at[pl.ds(r0 + b * CHUNK, CHUNK)])

  pltpu.sync_copy(rows_hbm.at[pl.ds(wid * ch, ch)], rows_v)
  pltpu.sync_copy(ei_hbm.at[1, pl.ds(wid * ch, ch)], cols_v)

  def _gathers(g, bank):
    for j in range(NBUF):
      pltpu.async_copy(table_hbm.at[rows_v.at[g * NBUF + j]],
                       gbuf.at[bank, j], sem_g)

  def _gathers_wait(g, bank):
    for j in range(NBUF):
      pltpu.make_async_copy(table_hbm.at[rows_v.at[g * NBUF + j]],
                            gbuf.at[bank, j], sem_g).wait()

  def _scatters(g, bank):
    for j in range(NBUF):
      pltpu.async_copy(gbuf.at[bank, j],
                       acc_sp.at[cols_v.at[g * NBUF + j]], sem_s, add=True)

  def _scatters_wait(g, bank):
    for j in range(NBUF):
      pltpu.make_async_copy(gbuf.at[bank, j],
                            acc_sp.at[cols_v.at[g * NBUF + j]], sem_s).wait()

  npairs = ch // NBUF // 2
  plsc.subcore_barrier()
  _gathers(0, 0)

  def _pair(p, c):
    a = 2 * p
    _gathers(a + 1, 1)
    _gathers_wait(a, 0)
    _scatters(a, 0)
    _scatters_wait(a, 0)

    @pl.when(p < npairs - 1)
    def _():
      _gathers(a + 2, 0)

    _gathers_wait(a + 1, 1)
    _scatters(a + 1, 1)
    _scatters_wait(a + 1, 1)
    return c
  lax.fori_loop(0, npairs, _pair, 0)

  plsc.subcore_barrier()
  pltpu.sync_copy(acc_sp.at[pl.ds(r0, rpt)],
                  part_hbm.at[cid, pl.ds(r0, rpt)])


def _mm1_body(x_ref, w_ref, y_ref):
  y_ref[...] = jnp.dot(x_ref[...], w_ref[...],
                       preferred_element_type=jnp.float32)


def _scale_body(y_ref, disb_ref, t_ref):
  t_ref[...] = disb_ref[...] * y_ref[...]


def _tc2_body(part_ref, t1_ref, disb_ref, b1_ref, w2_ref, t2_ref):
  agg = disb_ref[...] * (part_ref[0] + part_ref[1] + t1_ref[...])
  h = jnp.maximum(agg + b1_ref[...], 0.0)
  y2 = jnp.dot(h, w2_ref[...], preferred_element_type=jnp.float32)
  t2_ref[...] = disb_ref[...] * y2


def _tc3_body(fw, part_ref, t2_ref, disb_ref, b2_ref, out_ref):
  o = disb_ref[...] * (part_ref[0] + part_ref[1] + t2_ref[...]) + b2_ref[...]
  for h in range(2):
    oh = o[:, h * fw:(h + 1) * fw]
    m = jnp.max(oh, axis=-1, keepdims=True)
    lse = jnp.log(jnp.sum(jnp.exp(oh - m), axis=-1, keepdims=True)) + m
    out_ref[:, pl.ds(h * fw, fw)] = oh - lse


def kernel(x, edge_index, W1, b1, W2, b2):
  n, f_in = x.shape
  h_dim = W1.shape[1]
  c_dim = W2.shape[1]
  e = edge_index.shape[1]

  npad = ((n + 511) // 512 + (1 if n % 512 == 0 else 0)) * 512
  ch = -(-e // (NW * CHUNK))
  ch = -(-ch // 8) * 8
  e_pad = ch * NW * CHUNK
  blk = 1024
  blk2 = blk // 2
  grid = npad // blk

  ei_pad = jnp.pad(edge_index, ((0, 0), (0, e_pad - e)))
  ei_pad = ei_pad.reshape(2, e_pad // CHUNK, CHUNK)

  mesh = plsc.VectorSubcoreMesh(core_axis_name="c", subcore_axis_name="s")
  scp = pltpu.CompilerParams(use_tc_tiling_on_sc=False)

  deg_call = pl.kernel(
      functools.partial(_deg_body, n, npad, ch),
      out_type=(
          jax.ShapeDtypeStruct((NC, 1, npad), jnp.float32),
          jax.ShapeDtypeStruct((e_pad // CHUNK, CHUNK), jnp.int32),
      ),
      mesh=mesh,
      scratch_types=[
          pltpu.VMEM((ch, CHUNK), jnp.int32),
          pltpu.VMEM((ch, CHUNK), jnp.int32),
          pltpu.VMEM((ch, CHUNK), jnp.float32),
          pltpu.VMEM((npad // NS,), jnp.float32),
          pltpu.VMEM_SHARED((npad,), jnp.float32),
          pltpu.SemaphoreType.DMA,
      ],
      compiler_params=scp,
  )

  def spmm_call(fw):
    return pl.kernel(
        functools.partial(_spmm_body, npad, ch, fw),
        out_type=jax.ShapeDtypeStruct((NC, npad, fw), jnp.float32),
        mesh=mesh,
        scratch_types=[
            pltpu.VMEM((ch, CHUNK), jnp.int32),
            pltpu.VMEM((ch, CHUNK), jnp.int32),
            pltpu.VMEM((2, NBUF, CHUNK, fw), jnp.float32),
            pltpu.VMEM_SHARED((npad, fw), jnp.float32),
            pltpu.SemaphoreType.DMA,
            pltpu.SemaphoreType.DMA,
        ],
        compiler_params=scp,
    )

  spmm1 = spmm_call(h_dim)
  spmm2 = spmm1 if c_dim == h_dim else spmm_call(c_dim)

  deg_part, rows2d = deg_call(ei_pad)

  deg = deg_part[0, 0] + deg_part[1, 0] + 1.0
  dis = jnp.where(jnp.arange(npad) < n, lax.rsqrt(deg), 0.0)
  disb = jnp.repeat(dis.reshape(npad // 2, 2), h_dim, axis=1)

  y1 = pl.pallas_call(
      _mm1_body,
      grid=(grid,),
      in_specs=[
          pl.BlockSpec((blk, f_in), lambda i: (i, 0)),
          pl.BlockSpec((f_in, h_dim), lambda i: (0, 0)),
      ],
      out_specs=pl.BlockSpec((blk, h_dim), lambda i: (i, 0)),
      out_shape=jax.ShapeDtypeStruct((npad, h_dim), jnp.float32),
  )(x, W1)

  pk = (blk2, 2 * h_dim)
  t1p = pl.pallas_call(
      _scale_body,
      grid=(grid,),
      in_specs=[
          pl.BlockSpec(pk, lambda i: (i, 0)),
          pl.BlockSpec(pk, lambda i: (i, 0)),
      ],
      out_specs=pl.BlockSpec(pk, lambda i: (i, 0)),
      out_shape=jax.ShapeDtypeStruct((npad // 2, 2 * h_dim), jnp.float32),
  )(y1.reshape(npad // 2, 2 * h_dim), disb)

  t1 = t1p.reshape(npad, h_dim)
  part1 = spmm1(t1, rows2d, ei_pad)

  w2d = jnp.zeros((2 * h_dim, 2 * c_dim), jnp.float32)
  w2d = w2d.at[:h_dim, :c_dim].set(W2).at[h_dim:, c_dim:].set(W2)
  b1p = jnp.concatenate([b1, b1]).reshape(1, 2 * h_dim)
  b2p = jnp.concatenate([b2, b2]).reshape(1, 2 * c_dim)

  t2p = pl.pallas_call(
      _tc2_body,
      grid=(grid,),
      in_specs=[
          pl.BlockSpec((NC, blk2, 2 * h_dim), lambda i: (0, i, 0)),
          pl.BlockSpec(pk, lambda i: (i, 0)),
          pl.BlockSpec(pk, lambda i: (i, 0)),
          pl.BlockSpec((1, 2 * h_dim), lambda i: (0, 0)),
          pl.BlockSpec((2 * h_dim, 2 * c_dim), lambda i: (0, 0)),
      ],
      out_specs=pl.BlockSpec((blk2, 2 * c_dim), lambda i: (i, 0)),
      out_shape=jax.ShapeDtypeStruct((npad // 2, 2 * c_dim), jnp.float32),
  )(part1.reshape(NC, npad // 2, 2 * h_dim), t1p, disb, b1p, w2d)

  part2 = spmm2(t2p.reshape(npad, c_dim), rows2d, ei_pad)

  outp = pl.pallas_call(
      functools.partial(_tc3_body, c_dim),
      grid=(grid,),
      in_specs=[
          pl.BlockSpec((NC, blk2, 2 * c_dim), lambda i: (0, i, 0)),
          pl.BlockSpec((blk2, 2 * c_dim), lambda i: (i, 0)),
          pl.BlockSpec((blk2, 2 * c_dim), lambda i: (i, 0)),
          pl.BlockSpec((1, 2 * c_dim), lambda i: (0, 0)),
      ],
      out_specs=pl.BlockSpec((blk2, 2 * c_dim), lambda i: (i, 0)),
      out_shape=jax.ShapeDtypeStruct((npad // 2, 2 * c_dim), jnp.float32),
  )(part2.reshape(NC, npad // 2, 2 * c_dim), t2p, disb, b2p)

  return outp.reshape(npad, c_dim)[:n]


# speedup vs baseline: 1.0443x; 1.0360x over previous
import functools

import jax
import jax.numpy as jnp
from jax import lax
from jax.experimental import pallas as pl
from jax.experimental.pallas import tpu as pltpu
from jax.experimental.pallas import tpu_sc as plsc

NC = 2
NS = 16
NW = NC * NS
CHUNK = 128
NBUF = 2


def _deg_body(n, npad, ch, ei_hbm, deg_hbm, rowsg_hbm,
              rows_v, cols_v, val_v, zeros_v, deg_sp, sem_s):
  cid = lax.axis_index("c")
  sid = lax.axis_index("s")
  wid = sid * NC + cid
  rpt = npad // NS
  dump = npad - n

  def _z(i, c):
    zeros_v[pl.ds(i * 16, 16)] = jnp.zeros((16,), jnp.float32)
    return c
  lax.fori_loop(0, rpt // 16, _z, 0)
  pltpu.sync_copy(zeros_v, deg_sp.at[pl.ds(sid * rpt, rpt)])

  pltpu.sync_copy(ei_hbm.at[0, pl.ds(wid * ch, ch)], rows_v)
  pltpu.sync_copy(ei_hbm.at[1, pl.ds(wid * ch, ch)], cols_v)
  plsc.subcore_barrier()

  lanes = lax.iota(jnp.int32, 16)

  def _chunk(k, c):
    for j in range(CHUNK // 16):
      sl = pl.ds(j * 16, 16)
      r = rows_v[k, sl]
      cc = cols_v[k, sl]
      m = r != cc
      base = (wid * ch + k) * CHUNK + j * 16
      spr = n + ((base + lanes) % dump)
      rows_v[k, sl] = jnp.where(m, r, spr)
      val_v[k, sl] = jnp.where(m, 1.0, 0.0)
    pltpu.async_copy(val_v.at[k], deg_sp.at[cols_v.at[k]], sem_s, add=True)
    return c
  lax.fori_loop(0, ch, _chunk, 0)

  pltpu.sync_copy(rows_v, rowsg_hbm.at[pl.ds(wid * ch, ch)])

  def _drain(k, c):
    pltpu.make_async_copy(val_v.at[k], deg_sp.at[cols_v.at[k]], sem_s).wait()
    return c
  lax.fori_loop(0, ch, _drain, 0)

  plsc.subcore_barrier()
  pltpu.sync_copy(deg_sp.at[pl.ds(sid * rpt, rpt)],
                  deg_hbm.at[cid, 0, pl.ds(sid * rpt, rpt)])


def _spmm_body(npad, ch, fw, table_hbm, rows_hbm, ei_hbm, part_hbm,
               rows_v, cols_v, gbuf, acc_sp, sem_g, sem_s):
  cid = lax.axis_index("c")
  sid = lax.axis_index("s")
  wid = sid * NC + cid
  rpt = npad // NS
  r0 = sid * rpt

  def _z(i, c):
    for j in range(fw // 16):
      gbuf[0, 0, i, pl.ds(j * 16, 16)] = jnp.zeros((16,), jnp.float32)
    return c
  lax.fori_loop(0, CHUNK, _z, 0)
  for b in range(rpt // CHUNK):
    pltpu.sync_copy(gbuf.at[0, 0], acc_sp.at[pl.ds(r0 + b * CHUNK, CHUNK)])

  pltpu.sync_copy(rows_hbm.at[pl.ds(wid * ch, ch)], rows_v)
  pltpu.sync_copy(ei_hbm.at[1, pl.ds(wid * ch, ch)], cols_v)

  def _gathers(g, bank):
    for j in range(NBUF):
      pltpu.async_copy(table_hbm.at[rows_v.at[g * NBUF + j]],
                       gbuf.at[bank, j], sem_g)

  def _gathers_wait(g, bank):
    for j in range(NBUF):
      pltpu.make_async_copy(table_hbm.at[rows_v.at[g * NBUF + j]],
                            gbuf.at[bank, j], sem_g).wait()

  def _scatters(g, bank):
    for j in range(NBUF):
      pltpu.async_copy(gbuf.at[bank, j],
                       acc_sp.at[cols_v.at[g * NBUF + j]], sem_s, add=True)

  def _scatters_wait(g, bank):
    for j in range(NBUF):
      pltpu.make_async_copy(gbuf.at[bank, j],
                            acc_sp.at[cols_v.at[g * NBUF + j]], sem_s).wait()

  npairs = ch // NBUF // 2
  plsc.subcore_barrier()
  _gathers(0, 0)

  def _pair(p, c):
    a = 2 * p
    _gathers(a + 1, 1)
    _gathers_wait(a, 0)
    _scatters(a, 0)
    _scatters_wait(a, 0)

    @pl.when(p < npairs - 1)
    def _():
      _gathers(a + 2, 0)

    _gathers_wait(a + 1, 1)
    _scatters(a + 1, 1)
    _scatters_wait(a + 1, 1)
    return c
  lax.fori_loop(0, npairs, _pair, 0)

  plsc.subcore_barrier()
  pltpu.sync_copy(acc_sp.at[pl.ds(r0, rpt)],
                  part_hbm.at[cid, pl.ds(r0, rpt)])


def _mm1_body(x_ref, w_ref, y_ref):
  y_ref[...] = jnp.dot(x_ref[...], w_ref[...],
                       preferred_element_type=jnp.float32)


def _scale_body(y_ref, disb_ref, t_ref):
  t_ref[...] = disb_ref[...] * y_ref[...]


def _tc2_body(part_ref, t1_ref, disb_ref, b1_ref, w2_ref, t2_ref):
  agg = disb_ref[...] * (part_ref[0] + part_ref[1] + t1_ref[...])
  h = jnp.maximum(agg + b1_ref[...], 0.0)
  y2 = jnp.dot(h, w2_ref[...], preferred_element_type=jnp.float32)
  t2_ref[...] = disb_ref[...] * y2


def _tc3_body(fw, part_ref, t2_ref, disb_ref, b2_ref, out_ref):
  o = disb_ref[...] * (part_ref[0] + part_ref[1] + t2_ref[...]) + b2_ref[...]
  for h in range(2):
    oh = o[:, h * fw:(h + 1) * fw]
    m = jnp.max(oh, axis=-1, keepdims=True)
    lse = jnp.log(jnp.sum(jnp.exp(oh - m), axis=-1, keepdims=True)) + m
    out_ref[:, pl.ds(h * fw, fw)] = oh - lse


def kernel(x, edge_index, W1, b1, W2, b2):
  n, f_in = x.shape
  h_dim = W1.shape[1]
  c_dim = W2.shape[1]
  e = edge_index.shape[1]

  npad = ((n + 511) // 512 + (1 if n % 512 == 0 else 0)) * 512
  ch = -(-e // (NW * CHUNK))
  ch = -(-ch // 8) * 8
  e_pad = ch * NW * CHUNK
  blk = 2048
  blk2 = blk // 2
  grid = npad // blk

  ei_pad = jnp.pad(edge_index, ((0, 0), (0, e_pad - e)))
  ei_pad = ei_pad.reshape(2, e_pad // CHUNK, CHUNK)

  mesh = plsc.VectorSubcoreMesh(core_axis_name="c", subcore_axis_name="s")
  scp = pltpu.CompilerParams(use_tc_tiling_on_sc=False)

  deg_call = pl.kernel(
      functools.partial(_deg_body, n, npad, ch),
      out_type=(
          jax.ShapeDtypeStruct((NC, 1, npad), jnp.float32),
          jax.ShapeDtypeStruct((e_pad // CHUNK, CHUNK), jnp.int32),
      ),
      mesh=mesh,
      scratch_types=[
          pltpu.VMEM((ch, CHUNK), jnp.int32),
          pltpu.VMEM((ch, CHUNK), jnp.int32),
          pltpu.VMEM((ch, CHUNK), jnp.float32),
          pltpu.VMEM((npad // NS,), jnp.float32),
          pltpu.VMEM_SHARED((npad,), jnp.float32),
          pltpu.SemaphoreType.DMA,
      ],
      compiler_params=scp,
  )

  def spmm_call(fw):
    return pl.kernel(
        functools.partial(_spmm_body, npad, ch, fw),
        out_type=jax.ShapeDtypeStruct((NC, npad, fw), jnp.float32),
        mesh=mesh,
        scratch_types=[
            pltpu.VMEM((ch, CHUNK), jnp.int32),
            pltpu.VMEM((ch, CHUNK), jnp.int32),
            pltpu.VMEM((2, NBUF, CHUNK, fw), jnp.float32),
            pltpu.VMEM_SHARED((npad, fw), jnp.float32),
            pltpu.SemaphoreType.DMA,
            pltpu.SemaphoreType.DMA,
        ],
        compiler_params=scp,
    )

  spmm1 = spmm_call(h_dim)
  spmm2 = spmm1 if c_dim == h_dim else spmm_call(c_dim)

  deg_part, rows2d = deg_call(ei_pad)

  deg = deg_part[0, 0] + deg_part[1, 0] + 1.0
  dis = jnp.where(jnp.arange(npad) < n, lax.rsqrt(deg), 0.0)
  disb = jnp.repeat(dis.reshape(npad // 2, 2), h_dim, axis=1)

  y1 = pl.pallas_call(
      _mm1_body,
      grid=(grid,),
      in_specs=[
          pl.BlockSpec((blk, f_in), lambda i: (i, 0)),
          pl.BlockSpec((f_in, h_dim), lambda i: (0, 0)),
      ],
      out_specs=pl.BlockSpec((blk, h_dim), lambda i: (i, 0)),
      out_shape=jax.ShapeDtypeStruct((npad, h_dim), jnp.float32),
  )(x, W1)

  pk = (blk2, 2 * h_dim)
  t1p = pl.pallas_call(
      _scale_body,
      grid=(grid,),
      in_specs=[
          pl.BlockSpec(pk, lambda i: (i, 0)),
          pl.BlockSpec(pk, lambda i: (i, 0)),
      ],
      out_specs=pl.BlockSpec(pk, lambda i: (i, 0)),
      out_shape=jax.ShapeDtypeStruct((npad // 2, 2 * h_dim), jnp.float32),
  )(y1.reshape(npad // 2, 2 * h_dim), disb)

  t1 = t1p.reshape(npad, h_dim)
  part1 = spmm1(t1, rows2d, ei_pad)

  w2d = jnp.zeros((2 * h_dim, 2 * c_dim), jnp.float32)
  w2d = w2d.at[:h_dim, :c_dim].set(W2).at[h_dim:, c_dim:].set(W2)
  b1p = jnp.concatenate([b1, b1]).reshape(1, 2 * h_dim)
  b2p = jnp.concatenate([b2, b2]).reshape(1, 2 * c_dim)

  t2p = pl.pallas_call(
      _tc2_body,
      grid=(grid,),
      in_specs=[
          pl.BlockSpec((NC, blk2, 2 * h_dim), lambda i: (0, i, 0)),
          pl.BlockSpec(pk, lambda i: (i, 0)),
          pl.BlockSpec(pk, lambda i: (i, 0)),
          pl.BlockSpec((1, 2 * h_dim), lambda i: (0, 0)),
          pl.BlockSpec((2 * h_dim, 2 * c_dim), lambda i: (0, 0)),
      ],
      out_specs=pl.BlockSpec((blk2, 2 * c_dim), lambda i: (i, 0)),
      out_shape=jax.ShapeDtypeStruct((npad // 2, 2 * c_dim), jnp.float32),
  )(part1.reshape(NC, npad // 2, 2 * h_dim), t1p, disb, b1p, w2d)

  part2 = spmm2(t2p.reshape(npad, c_dim), rows2d, ei_pad)

  outp = pl.pallas_call(
      functools.partial(_tc3_body, c_dim),
      grid=(grid,),
      in_specs=[
          pl.BlockSpec((NC, blk2, 2 * c_dim), lambda i: (0, i, 0)),
          pl.BlockSpec((blk2, 2 * c_dim), lambda i: (i, 0)),
          pl.BlockSpec((blk2, 2 * c_dim), lambda i: (i, 0)),
          pl.BlockSpec((1, 2 * c_dim), lambda i: (0, 0)),
      ],
      out_specs=pl.BlockSpec((blk2, 2 * c_dim), lambda i: (i, 0)),
      out_shape=jax.ShapeDtypeStruct((npad // 2, 2 * c_dim), jnp.float32),
  )(part2.reshape(NC, npad // 2, 2 * c_dim), t2p, disb, b2p)

  return outp.reshape(npad, c_dim)[:n]
